# baseline (speedup 1.0000x reference)
import functools

import jax
import jax.numpy as jnp
from jax import lax
from jax.experimental import pallas as pl
from jax.experimental.pallas import tpu as pltpu
from jax.experimental.pallas import tpu_sc as plsc

NC = 2
NS = 16
NW = NC * NS


def _pick_chunk(per_w, d):
    budget = 400 * 1024 // (d * 4)
    best = 8
    for b in range(8, per_w + 1, 8):
        if per_w % b == 0 and b <= budget:
            best = b
    return best


@functools.partial(jax.jit, static_argnames=("chunk",))
def _sc_gather(table, idx, chunk):
    E = idx.shape[0]
    D = table.shape[1]
    per_w = E // NW
    n_iter = per_w // chunk
    mesh = plsc.VectorSubcoreMesh(core_axis_name="c", subcore_axis_name="s")

    @functools.partial(
        pl.kernel,
        out_type=jax.ShapeDtypeStruct((E, D), jnp.float32),
        mesh=mesh,
        scratch_types=[
            pltpu.VMEM((chunk,), jnp.int32),
            pltpu.VMEM((chunk, D), jnp.float32),
            pltpu.SemaphoreType.DMA,
        ],
        compiler_params=pltpu.CompilerParams(use_tc_tiling_on_sc=False),
    )
    def k(table_hbm, idx_hbm, out_hbm, idx_v, rows_v, sem):
        wid = lax.axis_index("s") * NC + lax.axis_index("c")
        base = wid * per_w

        def body(i, carry):
            off = base + i * chunk
            pltpu.sync_copy(idx_hbm.at[pl.ds(off, chunk)], idx_v)
            pltpu.async_copy(table_hbm.at[idx_v], rows_v, sem).wait()
            pltpu.sync_copy(rows_v, out_hbm.at[pl.ds(off, chunk)])
            return carry

        lax.fori_loop(0, n_iter, body, 0)

    return k(table, idx)


def _gather_rows(table, idx):
    D = table.shape[1]
    pad = (-D) % 16
    if pad:
        table = jnp.concatenate(
            [table, jnp.zeros((table.shape[0], pad), table.dtype)], axis=1)
    per_w = idx.shape[0] // NW
    out = _sc_gather(table, idx, _pick_chunk(per_w, D + pad))
    return out[:, :D] if pad else out


def _transformer_conv(x, src, dst, edge_attr, p, H, C, concat):
    N = x.shape[0]
    q = x @ p['Wq'].T + p['bq']
    k = x @ p['Wk'].T + p['bk']
    v = x @ p['Wv'].T + p['bv']
    if not concat:
        return _conv_mean(x, src, dst, edge_attr, p, q, k, v, H, C)
    e = (edge_attr @ p['We'].T).reshape(-1, H, C)
    kv_t = jnp.concatenate([k, v], axis=1)
    kv = _gather_rows(kv_t, src)
    k_j = kv[:, :H * C].reshape(-1, H, C) + e
    q_i = _gather_rows(q, dst).reshape(-1, H, C)
    alpha = jnp.sum(q_i * k_j, axis=-1) / jnp.sqrt(float(C))
    ex = jnp.exp(alpha)
    msg = (kv[:, H * C:].reshape(-1, H, C) + e) * ex[:, :, None]
    packed = jnp.concatenate([msg.reshape(-1, H * C), ex], axis=1)
    seg = jax.ops.segment_sum(packed, dst, num_segments=N)
    out = seg[:, :H * C] / (seg[:, H * C:] + 1e-16).repeat(C, axis=1)
    out = out if concat else out.reshape(N, H, C).mean(axis=1)
    x_r = x @ p['Ws'].T + p['bs']
    beta = jax.nn.sigmoid(
        jnp.concatenate([out, x_r, out - x_r], axis=-1) @ p['Wb'].T)
    return beta * x_r + (1.0 - beta) * out


def _conv_mean(x, src, dst, edge_attr, p, q, k, v, H, C):
    N = x.shape[0]
    E = src.shape[0]
    HC = H * C
    Wer = p['We'].reshape(H, C, -1)
    qe = jnp.einsum('nhc,hcd->nhd', q.reshape(N, H, C), Wer).reshape(N, -1)
    kv = _gather_rows(jnp.concatenate([k, v], axis=1), src)
    qq = _gather_rows(jnp.concatenate([q, qe], axis=1), dst)
    k_j = kv[:, :HC].reshape(E, H, C)
    q_i = qq[:, :HC].reshape(E, H, C)
    qe_i = qq[:, HC:].reshape(E, H, -1)
    alpha = (jnp.sum(q_i * k_j, axis=-1)
             + jnp.einsum('ed,ehd->eh', edge_attr, qe_i)) / jnp.sqrt(float(C))
    ex = jnp.exp(alpha)
    den = jax.ops.segment_sum(ex, dst, num_segments=N)
    den_t = jnp.pad(den, ((0, 0), (0, 128 - H)))
    al = ex / (_gather_rows(den_t, dst)[:, :H] + 1e-16)
    wv = jnp.einsum('eh,ehc->ec', al, kv[:, HC:].reshape(E, H, C))
    wea = (al[:, :, None] * edge_attr[:, None, :]).reshape(E, -1)
    seg = jax.ops.segment_sum(jnp.concatenate([wv, wea], axis=1),
                              dst, num_segments=N)
    out = (seg[:, :C]
           + jnp.einsum('nhd,hcd->nc', seg[:, C:].reshape(N, H, -1), Wer)) / H
    x_r = x @ p['Ws'].T + p['bs']
    beta = jax.nn.sigmoid(
        jnp.concatenate([out, x_r, out - x_r], axis=-1) @ p['Wb'].T)
    return beta * x_r + (1.0 - beta) * out


def _layer_norm(x, g, b):
    mu = x.mean(axis=-1, keepdims=True)
    var = x.var(axis=-1, keepdims=True)
    return (x - mu) / jnp.sqrt(var + 1e-5) * g + b


def kernel(x, edge_index, edge_attr, params):
    src = edge_index[0].astype(jnp.int32)
    dst = edge_index[1].astype(jnp.int32)
    h = jax.nn.relu(_layer_norm(
        _transformer_conv(x, src, dst, edge_attr, params['conv0'], 8, 16, True),
        params['ln0_g'], params['ln0_b']))
    h = jax.nn.relu(_layer_norm(
        _transformer_conv(h, src, dst, edge_attr, params['conv1'], 8, 16, True),
        params['ln1_g'], params['ln1_b']))
    h = _transformer_conv(h, src, dst, edge_attr, params['conv2'], 8, 64, False)
    W = params['lin_W']
    eaW3 = edge_attr @ W[:, 128:].T + params['lin_b']
    hW = jnp.concatenate([h @ W[:, :64].T, h @ W[:, 64:128].T], axis=1)
    hWd = _gather_rows(hW, dst)
    hWs = _gather_rows(hW, src)
    return hWd[:, :64] + hWs[:, 64:] + eaW3


# speedup vs baseline: 1.2014x; 1.2014x over previous
import functools

import jax
import jax.numpy as jnp
from jax import lax
from jax.experimental import pallas as pl
from jax.experimental.pallas import tpu as pltpu
from jax.experimental.pallas import tpu_sc as plsc

NC = 2
NS = 16
NW = NC * NS


def _pick_chunk(per_w, d):
    budget = 400 * 1024 // (d * 4)
    best = 8
    for b in range(8, per_w + 1, 8):
        if per_w % b == 0 and b <= budget:
            best = b
    return best


@functools.partial(jax.jit, static_argnames=("chunk",))
def _sc_gather(table, idx, chunk):
    E = idx.shape[0]
    D = table.shape[1]
    per_w = E // NW
    n_iter = per_w // chunk
    mesh = plsc.VectorSubcoreMesh(core_axis_name="c", subcore_axis_name="s")

    @functools.partial(
        pl.kernel,
        out_type=jax.ShapeDtypeStruct((E, D), jnp.float32),
        mesh=mesh,
        scratch_types=[
            pltpu.VMEM((chunk,), jnp.int32),
            pltpu.VMEM((chunk, D), jnp.float32),
            pltpu.SemaphoreType.DMA,
        ],
    )
    def k(table_hbm, idx_hbm, out_hbm, idx_v, rows_v, sem):
        wid = lax.axis_index("s") * NC + lax.axis_index("c")
        base = wid * per_w

        def body(i, carry):
            off = base + i * chunk
            pltpu.sync_copy(idx_hbm.at[pl.ds(off, chunk)], idx_v)
            pltpu.async_copy(table_hbm.at[idx_v], rows_v, sem).wait()
            pltpu.sync_copy(rows_v, out_hbm.at[pl.ds(off, chunk)])
            return carry

        lax.fori_loop(0, n_iter, body, 0)

    return k(table, idx)


def _gather_rows(table, idx):
    D = table.shape[1]
    pad = (-D) % 16
    if pad:
        table = jnp.concatenate(
            [table, jnp.zeros((table.shape[0], pad), table.dtype)], axis=1)
    per_w = idx.shape[0] // NW
    out = _sc_gather(table, idx, _pick_chunk(per_w, D + pad))
    return out[:, :D] if pad else out


def _transformer_conv(x, src, dst, edge_attr, p, H, C, concat):
    N = x.shape[0]
    q = x @ p['Wq'].T + p['bq']
    k = x @ p['Wk'].T + p['bk']
    v = x @ p['Wv'].T + p['bv']
    if not concat:
        return _conv_mean(x, src, dst, edge_attr, p, q, k, v, H, C)
    e = (edge_attr @ p['We'].T).reshape(-1, H, C)
    kv_t = jnp.concatenate([k, v], axis=1)
    kv = _gather_rows(kv_t, src)
    k_j = kv[:, :H * C].reshape(-1, H, C) + e
    q_i = _gather_rows(q, dst).reshape(-1, H, C)
    alpha = jnp.sum(q_i * k_j, axis=-1) / jnp.sqrt(float(C))
    ex = jnp.exp(alpha)
    msg = (kv[:, H * C:].reshape(-1, H, C) + e) * ex[:, :, None]
    packed = jnp.concatenate([msg.reshape(-1, H * C), ex], axis=1)
    seg = jax.ops.segment_sum(packed, dst, num_segments=N)
    out = seg[:, :H * C] / (seg[:, H * C:] + 1e-16).repeat(C, axis=1)
    out = out if concat else out.reshape(N, H, C).mean(axis=1)
    x_r = x @ p['Ws'].T + p['bs']
    beta = jax.nn.sigmoid(
        jnp.concatenate([out, x_r, out - x_r], axis=-1) @ p['Wb'].T)
    return beta * x_r + (1.0 - beta) * out


def _conv_mean(x, src, dst, edge_attr, p, q, k, v, H, C):
    N = x.shape[0]
    E = src.shape[0]
    HC = H * C
    Wer = p['We'].reshape(H, C, -1)
    qe = jnp.einsum('nhc,hcd->nhd', q.reshape(N, H, C), Wer).reshape(N, -1)
    kv = _gather_rows(jnp.concatenate([k, v], axis=1), src)
    k_j = kv[:, :HC].reshape(E, H, C)
    q_i = _gather_rows(q, dst).reshape(E, H, C)
    qe_i = _gather_rows(qe, dst).reshape(E, H, -1)
    alpha = (jnp.sum(q_i * k_j, axis=-1)
             + jnp.einsum('ed,ehd->eh', edge_attr, qe_i)) / jnp.sqrt(float(C))
    ex = jnp.exp(alpha)
    den = jax.ops.segment_sum(ex, dst, num_segments=N)
    den_t = jnp.pad(den, ((0, 0), (0, 128 - H)))
    al = ex / (_gather_rows(den_t, dst)[:, :H] + 1e-16)
    wv = jnp.einsum('eh,ehc->ec', al, kv[:, HC:].reshape(E, H, C))
    wea = (al[:, :, None] * edge_attr[:, None, :]).reshape(E, -1)
    seg = jax.ops.segment_sum(jnp.concatenate([wv, wea], axis=1),
                              dst, num_segments=N)
    out = (seg[:, :C]
           + jnp.einsum('nhd,hcd->nc', seg[:, C:].reshape(N, H, -1), Wer)) / H
    x_r = x @ p['Ws'].T + p['bs']
    beta = jax.nn.sigmoid(
        jnp.concatenate([out, x_r, out - x_r], axis=-1) @ p['Wb'].T)
    return beta * x_r + (1.0 - beta) * out


def _layer_norm(x, g, b):
    mu = x.mean(axis=-1, keepdims=True)
    var = x.var(axis=-1, keepdims=True)
    return (x - mu) / jnp.sqrt(var + 1e-5) * g + b


def kernel(x, edge_index, edge_attr, params):
    src = edge_index[0].astype(jnp.int32)
    dst = edge_index[1].astype(jnp.int32)
    h = jax.nn.relu(_layer_norm(
        _transformer_conv(x, src, dst, edge_attr, params['conv0'], 8, 16, True),
        params['ln0_g'], params['ln0_b']))
    h = jax.nn.relu(_layer_norm(
        _transformer_conv(h, src, dst, edge_attr, params['conv1'], 8, 16, True),
        params['ln1_g'], params['ln1_b']))
    h = _transformer_conv(h, src, dst, edge_attr, params['conv2'], 8, 64, False)
    W = params['lin_W']
    eaW3 = edge_attr @ W[:, 128:].T + params['lin_b']
    hW = jnp.concatenate([h @ W[:, :64].T, h @ W[:, 64:128].T], axis=1)
    hWd = _gather_rows(hW, dst)
    hWs = _gather_rows(hW, src)
    return hWd[:, :64] + hWs[:, 64:] + eaW3
